# Initial kernel scaffold; baseline (speedup 1.0000x reference)
import jax
import jax.numpy as jnp
from jax.experimental import pallas as pl


def kernel(x, table):
    raise NotImplementedError("write your pallas kernel here")


# speedup vs baseline: 3.3056x; 3.3056x over previous
import functools

import jax
import jax.numpy as jnp
from jax import lax
from jax.experimental import pallas as pl
from jax.experimental.pallas import tpu as pltpu
from jax.experimental.pallas import tpu_sc as plsc

_NTOKEN = 100000
_EMB_DIM = 128
_B = 4096
_L = 50

_INFO = plsc.get_sparse_core_info()
_NC = _INFO.num_cores
_NS = _INFO.num_subcores
_NW = _NC * _NS

_N_ROWS = _B * _L
_B_PER_W = _N_ROWS // _NW
_CHUNK = 128
_NCH = _B_PER_W // _CHUNK
_NBUF = 5
_NGROUP = _NCH // _NBUF


def _emb_body(idx_hbm, table_hbm, out_hbm, idx_v, *rest):
    bufs = rest[:_NBUF]
    gsems = rest[_NBUF : 2 * _NBUF]
    osems = rest[2 * _NBUF : 3 * _NBUF]

    wid = lax.axis_index("s") * _NC + lax.axis_index("c")
    base = wid * _B_PER_W

    pltpu.sync_copy(idx_hbm.at[wid], idx_v)

    for b in range(_NBUF):
        pltpu.async_copy(table_hbm.at[idx_v.at[b]], bufs[b], gsems[b])

    @pl.loop(0, _NGROUP)
    def _group(g):
        for b in range(_NBUF):
            ch = g * _NBUF + b
            pltpu.make_async_copy(
                table_hbm.at[idx_v.at[b]], bufs[b], gsems[b]
            ).wait()
            pltpu.async_copy(
                bufs[b],
                out_hbm.at[pl.ds(base + ch * _CHUNK, _CHUNK)],
                osems[b],
            )
        for b in range(_NBUF):

            @pl.when(g + 1 < _NGROUP)
            def _():
                pltpu.make_async_copy(
                    bufs[b], out_hbm.at[pl.ds(0, _CHUNK)], osems[b]
                ).wait()
                nch = (g + 1) * _NBUF + b
                pltpu.async_copy(
                    table_hbm.at[idx_v.at[nch]], bufs[b], gsems[b]
                )

    for b in range(_NBUF):
        pltpu.make_async_copy(
            bufs[b], out_hbm.at[pl.ds(0, _CHUNK)], osems[b]
        ).wait()


@functools.partial(jax.jit, static_argnames=())
def _emb(idx, table):
    mesh = plsc.VectorSubcoreMesh(core_axis_name="c", subcore_axis_name="s")
    scratch = [pltpu.VMEM((_NCH, _CHUNK), jnp.int32)]
    scratch += [pltpu.VMEM((_CHUNK, _EMB_DIM), jnp.float32) for _ in range(_NBUF)]
    scratch += [pltpu.SemaphoreType.DMA for _ in range(2 * _NBUF)]
    run = pl.kernel(
        _emb_body,
        out_type=jax.ShapeDtypeStruct((_N_ROWS, _EMB_DIM), jnp.float32),
        mesh=mesh,
        scratch_types=scratch,
    )
    return run(idx, table)


def kernel(x, table):
    idx = jnp.asarray(x, jnp.int32).reshape(_NW, _NCH, _CHUNK)
    out = _emb(idx, table)
    return out.reshape(_B, _L, _EMB_DIM)


# speedup vs baseline: 5.9286x; 1.7935x over previous
import functools

import jax
import jax.numpy as jnp
from jax import lax
from jax.experimental import pallas as pl
from jax.experimental.pallas import tpu as pltpu
from jax.experimental.pallas import tpu_sc as plsc

_NTOKEN = 100000
_EMB_DIM = 128
_B = 4096
_L = 50
_LP = 56

_INFO = plsc.get_sparse_core_info()
_NC = _INFO.num_cores
_NS = _INFO.num_subcores
_NW = _NC * _NS

_B_PER_W = _B // _NW
_NBUF = 8
_NGROUP = _B_PER_W // _NBUF


def _emb_body(idx_hbm, table_hbm, out_hbm, idx_v, *rest):
    bufs = rest[:_NBUF]
    gsems = rest[_NBUF : 2 * _NBUF]
    osems = rest[2 * _NBUF : 3 * _NBUF]

    wid = lax.axis_index("s") * _NC + lax.axis_index("c")
    base = wid * _B_PER_W

    pltpu.sync_copy(idx_hbm.at[wid], idx_v)

    for b in range(_NBUF):
        pltpu.async_copy(
            table_hbm.at[idx_v.at[b, pl.ds(0, _L)]], bufs[b], gsems[b]
        )

    @pl.loop(0, _NGROUP)
    def _group(g):
        for b in range(_NBUF):
            ch = g * _NBUF + b
            pltpu.make_async_copy(
                table_hbm.at[idx_v.at[0, pl.ds(0, _L)]], bufs[b], gsems[b]
            ).wait()
            pltpu.async_copy(bufs[b], out_hbm.at[base + ch], osems[b])
        for b in range(_NBUF):

            @pl.when(g + 1 < _NGROUP)
            def _():
                pltpu.make_async_copy(
                    bufs[b], out_hbm.at[base], osems[b]
                ).wait()
                nch = (g + 1) * _NBUF + b
                pltpu.async_copy(
                    table_hbm.at[idx_v.at[nch, pl.ds(0, _L)]], bufs[b], gsems[b]
                )

    for b in range(_NBUF):
        pltpu.make_async_copy(bufs[b], out_hbm.at[base], osems[b]).wait()


@jax.jit
def _emb(idx, table):
    mesh = plsc.VectorSubcoreMesh(core_axis_name="c", subcore_axis_name="s")
    scratch = [pltpu.VMEM((_B_PER_W, _LP), jnp.int32)]
    scratch += [pltpu.VMEM((_L, _EMB_DIM), jnp.float32) for _ in range(_NBUF)]
    scratch += [pltpu.SemaphoreType.DMA for _ in range(2 * _NBUF)]
    run = pl.kernel(
        _emb_body,
        out_type=jax.ShapeDtypeStruct((_B, _L, _EMB_DIM), jnp.float32),
        mesh=mesh,
        scratch_types=scratch,
    )
    return run(idx, table)


def kernel(x, table):
    idx = jnp.asarray(x, jnp.int32)
    idx = jnp.pad(idx, ((0, 0), (0, _LP - _L)))
    idx = idx.reshape(_NW, _B_PER_W, _LP)
    return _emb(idx, table)


# speedup vs baseline: 10.2737x; 1.7329x over previous
import jax
import jax.numpy as jnp
from jax import lax
from jax.experimental import pallas as pl
from jax.experimental.pallas import tpu as pltpu
from jax.experimental.pallas import tpu_sc as plsc

_NTOKEN = 100000
_EMB_DIM = 128
_B = 4096
_L = 50

_INFO = plsc.get_sparse_core_info()
_NC = _INFO.num_cores
_NS = _INFO.num_subcores
_NW = _NC * _NS

_B_PER_W = _B // _NW
_NBUF = 5
_NGROUP = _L // _NBUF


def _emb_body(idx_hbm, table_hbm, out_hbm, idx_v, *rest):
    bufs = rest[:_NBUF]
    gsems = rest[_NBUF : 2 * _NBUF]
    osems = rest[2 * _NBUF : 3 * _NBUF]

    wid = lax.axis_index("s") * _NC + lax.axis_index("c")
    base = wid * _B_PER_W

    pltpu.sync_copy(idx_hbm.at[wid], idx_v)

    for b in range(_NBUF):
        pltpu.async_copy(table_hbm.at[idx_v.at[b]], bufs[b], gsems[b])

    @pl.loop(0, _NGROUP)
    def _group(g):
        for b in range(_NBUF):
            ch = g * _NBUF + b
            pltpu.make_async_copy(
                table_hbm.at[idx_v.at[0]], bufs[b], gsems[b]
            ).wait()
            pltpu.async_copy(
                bufs[b], out_hbm.at[ch, pl.ds(base, _B_PER_W)], osems[b]
            )
        for b in range(_NBUF):

            @pl.when(g + 1 < _NGROUP)
            def _():
                pltpu.make_async_copy(
                    bufs[b], out_hbm.at[0, pl.ds(base, _B_PER_W)], osems[b]
                ).wait()
                nch = (g + 1) * _NBUF + b
                pltpu.async_copy(
                    table_hbm.at[idx_v.at[nch]], bufs[b], gsems[b]
                )

    for b in range(_NBUF):
        pltpu.make_async_copy(
            bufs[b], out_hbm.at[0, pl.ds(base, _B_PER_W)], osems[b]
        ).wait()


@jax.jit
def _emb(idx, table):
    mesh = plsc.VectorSubcoreMesh(core_axis_name="c", subcore_axis_name="s")
    scratch = [pltpu.VMEM((_L, _B_PER_W), jnp.int32)]
    scratch += [
        pltpu.VMEM((_B_PER_W, _EMB_DIM), jnp.float32) for _ in range(_NBUF)
    ]
    scratch += [pltpu.SemaphoreType.DMA for _ in range(2 * _NBUF)]
    run = pl.kernel(
        _emb_body,
        out_type=jax.ShapeDtypeStruct((_L, _B, _EMB_DIM), jnp.float32),
        mesh=mesh,
        scratch_types=scratch,
    )
    return run(idx, table)


def kernel(x, table):
    idx = jnp.asarray(x, jnp.int32).T.reshape(_L, _NW, _B_PER_W)
    idx = idx.transpose(1, 0, 2)
    out = _emb(idx, table)
    return out.transpose(1, 0, 2)
